# baseline (speedup 1.0000x reference)
import functools

import jax
import jax.numpy as jnp
from jax import lax
from jax.experimental import pallas as pl
from jax.experimental.pallas import tpu as pltpu
from jax.experimental.pallas import tpu_sc as plsc

_LANES = 16
_CB = 16
_W = 128


@functools.lru_cache(maxsize=None)
def _make_relayout(V, E):
    NV = 512
    grid = (pl.cdiv(V, NV),)

    def body(in_ref, out_ref):
        z = jnp.swapaxes(in_ref[...], 0, 1)
        out_ref[:, 0:E] = z

    return pl.pallas_call(
        body,
        grid=grid,
        in_specs=[pl.BlockSpec((E, NV), lambda i: (0, i))],
        out_specs=pl.BlockSpec((NV, _W), lambda i: (i, 0)),
        out_shape=jax.ShapeDtypeStruct((V, _W), jnp.float32),
    )


@functools.lru_cache(maxsize=None)
def _make_pool(B, L, E, V):
    info = plsc.get_sparse_core_info()
    nc, ns = info.num_cores, info.num_subcores
    nw = nc * ns
    bpw = B // nw
    nchunks = bpw // _CB
    ecols = E // _LANES
    mesh = plsc.VectorSubcoreMesh(core_axis_name="c", subcore_axis_name="s")

    @functools.partial(
        pl.kernel,
        mesh=mesh,
        compiler_params=pltpu.CompilerParams(use_tc_tiling_on_sc=True),
        out_type=jax.ShapeDtypeStruct((B, _W), jnp.float32),
        scratch_types=[
            pltpu.VMEM((_CB * L,), jnp.int32),
            pltpu.VMEM((_CB * L, _W), jnp.float32),
            pltpu.VMEM((_CB, _W), jnp.float32),
            pltpu.SemaphoreType.DMA,
        ],
    )
    def pool(xf_hbm, emb_hbm, out_hbm, idx_v, rows_v, acc_v, sem):
        wid = lax.axis_index("s") * nc + lax.axis_index("c")
        row0 = wid * bpw

        def chunk_body(ci, carry):
            base = row0 + ci * _CB
            pltpu.sync_copy(xf_hbm.at[pl.ds(base * L, _CB * L)], idx_v)
            pltpu.async_copy(emb_hbm.at[idx_v], rows_v, sem).wait()

            def row_body(b, carry2):
                def j_body(j, acc):
                    r = b * L + j
                    return tuple(
                        acc[c] + rows_v[r, pl.ds(c * _LANES, _LANES)]
                        for c in range(ecols)
                    )

                zero = jnp.zeros((_LANES,), jnp.float32)
                acc = lax.fori_loop(0, L, j_body, (zero,) * ecols)
                for c in range(ecols):
                    acc_v[b, pl.ds(c * _LANES, _LANES)] = acc[c]
                return carry2

            lax.fori_loop(0, _CB, row_body, 0)
            pltpu.sync_copy(acc_v, out_hbm.at[pl.ds(base, _CB)])
            return carry

        lax.fori_loop(0, nchunks, chunk_body, 0)

    return pool


@functools.lru_cache(maxsize=None)
def _make_mlp(B, E, H, O, L):
    bm = 2048

    def mlp_body(s_ref, w1_ref, b1_ref, w2_ref, b2_ref, o_ref):
        m = s_ref[:, 0:E] * (1.0 / L)
        h = lax.dot(m, w1_ref[...], precision=lax.Precision.HIGHEST)
        h = jnp.maximum(h + b1_ref[...], 0.0)
        o_ref[...] = (
            lax.dot(h, w2_ref[...], precision=lax.Precision.HIGHEST) + b2_ref[...]
        )

    return pl.pallas_call(
        mlp_body,
        grid=(B // bm,),
        in_specs=[
            pl.BlockSpec((bm, _W), lambda i: (i, 0)),
            pl.BlockSpec((E, H), lambda i: (0, 0)),
            pl.BlockSpec((1, H), lambda i: (0, 0)),
            pl.BlockSpec((H, O), lambda i: (0, 0)),
            pl.BlockSpec((1, O), lambda i: (0, 0)),
        ],
        out_specs=pl.BlockSpec((bm, O), lambda i: (i, 0)),
        out_shape=jax.ShapeDtypeStruct((B, O), jnp.float32),
    )


def kernel(x, emb, W1, b1, W2, b2):
    B, L = x.shape
    V, E = emb.shape
    H = W1.shape[1]
    O = W2.shape[1]
    table = _make_relayout(V, E)(emb.T)
    pooled = _make_pool(B, L, E, V)(x.reshape(-1), table)
    return _make_mlp(B, E, H, O, L)(
        pooled, W1, b1.reshape(1, H), W2, b2.reshape(1, O)
    )


# speedup vs baseline: 1.9792x; 1.9792x over previous
import functools

import jax
import jax.numpy as jnp
from jax import lax
from jax.experimental import pallas as pl
from jax.experimental.pallas import tpu as pltpu
from jax.experimental.pallas import tpu_sc as plsc

_LANES = 16
_CB = 8


@functools.lru_cache(maxsize=None)
def _make_pool(B, L, E, V):
    info = plsc.get_sparse_core_info()
    nc, ns = info.num_cores, info.num_subcores
    nw = nc * ns
    bpw = B // nw
    nchunks = bpw // _CB
    ecols = E // _LANES
    mesh = plsc.VectorSubcoreMesh(core_axis_name="c", subcore_axis_name="s")

    @functools.partial(
        pl.kernel,
        mesh=mesh,
        compiler_params=pltpu.CompilerParams(use_tc_tiling_on_sc=False),
        out_type=jax.ShapeDtypeStruct((B, E), jnp.float32),
        scratch_types=[
            pltpu.VMEM((bpw * L,), jnp.int32),
            pltpu.VMEM((_CB * L, E), jnp.float32),
            pltpu.VMEM((_CB * L, E), jnp.float32),
            pltpu.VMEM((bpw, E), jnp.float32),
            pltpu.SemaphoreType.DMA,
            pltpu.SemaphoreType.DMA,
        ],
    )
    def pool(xf_hbm, emb_hbm, out_hbm, idx_v, rows0, rows1, stage, s0, s1):
        wid = lax.axis_index("s") * nc + lax.axis_index("c")
        row0 = wid * bpw
        rows = [rows0, rows1]
        sems = [s0, s1]

        pltpu.sync_copy(xf_hbm.at[pl.ds(row0 * L, bpw * L)], idx_v)

        def start_gather(g, p):
            pltpu.async_copy(
                emb_hbm.at[idx_v.at[pl.ds(g * (_CB * L), _CB * L)]],
                rows[p], sems[p])

        def wait_gather(p):
            pltpu.make_async_copy(
                emb_hbm.at[idx_v.at[pl.ds(0, _CB * L)]],
                rows[p], sems[p]).wait()

        def reduce_chunk(g, p):
            rows_v = rows[p]

            def row_body(b, carry2):
                def j_body(j, acc):
                    r = b * L + j
                    return tuple(
                        acc[c] + rows_v[r, pl.ds(c * _LANES, _LANES)]
                        for c in range(ecols)
                    )

                zero = jnp.zeros((_LANES,), jnp.float32)
                acc = lax.fori_loop(0, L, j_body, (zero,) * ecols)
                w = g * _CB + b
                for c in range(ecols):
                    stage[w, pl.ds(c * _LANES, _LANES)] = acc[c]
                return carry2

            lax.fori_loop(0, _CB, row_body, 0)

        start_gather(0, 0)

        def pair(h, carry):
            g0 = 2 * h
            start_gather(g0 + 1, 1)
            wait_gather(0)
            reduce_chunk(g0, 0)

            @pl.when(h + 1 < nchunks // 2)
            def _():
                start_gather(g0 + 2, 0)

            wait_gather(1)
            reduce_chunk(g0 + 1, 1)
            return carry

        lax.fori_loop(0, nchunks // 2, pair, 0)
        pltpu.sync_copy(stage, out_hbm.at[pl.ds(row0, bpw)])

    return pool


@functools.lru_cache(maxsize=None)
def _make_mlp(B, E, H, O, L):
    bm = 2048

    def mlp_body(s_ref, w1_ref, b1_ref, w2_ref, b2_ref, o_ref):
        m = s_ref[...] * (1.0 / L)
        h = lax.dot(m, w1_ref[...], precision=lax.Precision.HIGHEST)
        h = jnp.maximum(h + b1_ref[...], 0.0)
        o_ref[...] = (
            lax.dot(h, w2_ref[...], precision=lax.Precision.HIGHEST) + b2_ref[...]
        )

    return pl.pallas_call(
        mlp_body,
        grid=(B // bm,),
        in_specs=[
            pl.BlockSpec((bm, E), lambda i: (i, 0)),
            pl.BlockSpec((E, H), lambda i: (0, 0)),
            pl.BlockSpec((1, H), lambda i: (0, 0)),
            pl.BlockSpec((H, O), lambda i: (0, 0)),
            pl.BlockSpec((1, O), lambda i: (0, 0)),
        ],
        out_specs=pl.BlockSpec((bm, O), lambda i: (i, 0)),
        out_shape=jax.ShapeDtypeStruct((B, O), jnp.float32),
    )


def kernel(x, emb, W1, b1, W2, b2):
    B, L = x.shape
    V, E = emb.shape
    H = W1.shape[1]
    O = W2.shape[1]
    pooled = _make_pool(B, L, E, V)(x.reshape(-1), emb)
    return _make_mlp(B, E, H, O, L)(
        pooled, W1, b1.reshape(1, H), W2, b2.reshape(1, O)
    )
